# baseline (speedup 1.0000x reference)
import jax
import jax.numpy as jnp
from jax import lax
from jax.experimental import pallas as pl
from jax.experimental.pallas import tpu as pltpu
from jax.experimental.pallas import tpu_sc as plsc

NC = 2
NS = 16
NW = NC * NS
L = 16
CHUNK = 128


def _body(user_hbm, item_hbm, ue_hbm, ie_hbm, ub_hbm, ib_hbm, out_hbm,
          idx_u, idx_i, u_rows, i_rows, ub_v, ib_v, out_v, sem):
    nchunk, _ = idx_u.shape
    bpw = nchunk * CHUNK
    wid = lax.axis_index("s") * NC + lax.axis_index("c")
    base_chunk = wid * nchunk

    pltpu.sync_copy(user_hbm.at[pl.ds(base_chunk, nchunk)], idx_u)
    pltpu.sync_copy(item_hbm.at[pl.ds(base_chunk, nchunk)], idx_i)

    copies = []
    for j in range(nchunk):
        sl = pl.ds(j * CHUNK, CHUNK)
        copies.append(pltpu.async_copy(ue_hbm.at[idx_u.at[j]], u_rows.at[sl], sem))
        copies.append(pltpu.async_copy(ie_hbm.at[idx_i.at[j]], i_rows.at[sl], sem))
        copies.append(pltpu.async_copy(ub_hbm.at[idx_u.at[j]], ub_v.at[sl], sem))
        copies.append(pltpu.async_copy(ib_hbm.at[idx_i.at[j]], ib_v.at[sl], sem))
    for c in copies:
        c.wait()

    lane = lax.broadcasted_iota(jnp.int32, (L,), 0)

    def group(g, carry):
        out = jnp.zeros((L,), jnp.float32)
        for r in range(L):
            b = g * L + r
            u0 = u_rows[b, pl.ds(0, L)]
            u1 = u_rows[b, pl.ds(L, L)]
            i0 = i_rows[b, pl.ds(0, L)]
            i1 = i_rows[b, pl.ds(L, L)]
            s = jnp.sum(u0 * i0 + u1 * i1)
            out = jnp.where(lane == r, s, out)
        sl = pl.ds(g * L, L)
        x = out + ub_v[sl] + ib_v[sl]
        out_v[sl] = 4.0 / (1.0 + jnp.exp(-x)) + 1.0
        return carry

    lax.fori_loop(0, bpw // L, group, 0)

    pltpu.sync_copy(out_v, out_hbm.at[pl.ds(wid * bpw, bpw)])


def kernel(user, item, user_emb_w, item_emb_w, user_bias_w, item_bias_w):
    batch = user.shape[0]
    d = user_emb_w.shape[1]
    bpw = batch // NW
    nchunk = bpw // CHUNK
    assert bpw * NW == batch and nchunk * CHUNK == bpw

    mesh = plsc.VectorSubcoreMesh(
        core_axis_name="c", subcore_axis_name="s",
        num_cores=NC, num_subcores=NS)
    f = pl.kernel(
        _body,
        out_type=jax.ShapeDtypeStruct((batch,), jnp.float32),
        mesh=mesh,
        compiler_params=pltpu.CompilerParams(
            needs_layout_passes=False, use_tc_tiling_on_sc=False),
        scratch_types=[
            pltpu.VMEM((nchunk, CHUNK), jnp.int32),
            pltpu.VMEM((nchunk, CHUNK), jnp.int32),
            pltpu.VMEM((bpw, d), jnp.float32),
            pltpu.VMEM((bpw, d), jnp.float32),
            pltpu.VMEM((bpw,), jnp.float32),
            pltpu.VMEM((bpw,), jnp.float32),
            pltpu.VMEM((bpw,), jnp.float32),
            pltpu.SemaphoreType.DMA,
        ],
    )
    user2 = user.reshape(batch // CHUNK, CHUNK)
    item2 = item.reshape(batch // CHUNK, CHUNK)
    ub1 = user_bias_w.reshape(-1)
    ib1 = item_bias_w.reshape(-1)
    return f(user2, item2, user_emb_w, item_emb_w, ub1, ib1)


# speedup vs baseline: 2.2976x; 2.2976x over previous
import jax
import jax.numpy as jnp
from jax import lax
from jax.experimental import pallas as pl
from jax.experimental.pallas import tpu as pltpu
from jax.experimental.pallas import tpu_sc as plsc

NC = 2
NS = 16
NW = NC * NS
L = 16
CHUNK = 128
GPC = CHUNK // L


def _body(user_hbm, item_hbm, ue_hbm, ie_hbm, ub_hbm, ib_hbm, out_hbm,
          idx_vu, idx_vi, u_t, i_t, ub_v, ib_v, out_v, sem_u, sem_i, sem_b):
    nchunk, _ = idx_vu.shape
    bpw = nchunk * CHUNK
    d = u_t.shape[1]
    ngroups = bpw // L
    wid = lax.axis_index("s") * NC + lax.axis_index("c")
    base_chunk = wid * nchunk

    pltpu.sync_copy(user_hbm.at[pl.ds(base_chunk, nchunk)], idx_vu)
    pltpu.sync_copy(item_hbm.at[pl.ds(base_chunk, nchunk)], idx_vi)
    bias_copies = []
    for j in range(nchunk):
        sl = pl.ds(j * CHUNK, CHUNK)
        bias_copies.append(
            pltpu.async_copy(ub_hbm.at[idx_vu.at[j]], ub_v.at[sl], sem_b))
        bias_copies.append(
            pltpu.async_copy(ib_hbm.at[idx_vi.at[j]], ib_v.at[sl], sem_b))

    def load_group_idx(g):
        j = g // GPC
        k = (g % GPC) * L
        return idx_vu[j, pl.ds(k, L)], idx_vi[j, pl.ds(k, L)]

    def fire(ru, ri, buf):
        ctu = pl.multiple_of((ru // CHUNK) * CHUNK, CHUNK)
        cti = pl.multiple_of((ri // CHUNK) * CHUNK, CHUNK)
        pltpu.async_copy(ue_hbm.at[:, pl.ds(ctu, CHUNK)], u_t.at[buf], sem_u)
        pltpu.async_copy(ie_hbm.at[:, pl.ds(cti, CHUNK)], i_t.at[buf], sem_i)

    def drain(buf):
        pltpu.make_async_copy(
            ue_hbm.at[:, pl.ds(0, CHUNK)], u_t.at[buf], sem_u).wait()
        pltpu.make_async_copy(
            ie_hbm.at[:, pl.ds(0, CHUNK)], i_t.at[buf], sem_i).wait()

    lane = lax.broadcasted_iota(jnp.int32, (L,), 0)
    d_lo = lax.broadcasted_iota(jnp.int32, (L,), 0)
    d_hi = d_lo + L

    iv_u0, iv_i0 = load_group_idx(0)
    fire(iv_u0[0], iv_i0[0], 0)

    def group(g, carry):
        iv_u, iv_i = load_group_idx(g)
        out = jnp.zeros((L,), jnp.float32)
        for t in range(L):
            if t < L - 1:
                fire(iv_u[t + 1], iv_i[t + 1], (t + 1) % 2)
            else:
                @pl.when(g + 1 < ngroups)
                def _():
                    iv_un, iv_in = load_group_idx(g + 1)
                    fire(iv_un[0], iv_in[0], 0)
            drain(t % 2)
            cu = jnp.full((L,), iv_u[t] % CHUNK, jnp.int32)
            ci = jnp.full((L,), iv_i[t] % CHUNK, jnp.int32)
            u0 = plsc.load_gather(u_t.at[t % 2], [d_lo, cu])
            u1 = plsc.load_gather(u_t.at[t % 2], [d_hi, cu])
            i0 = plsc.load_gather(i_t.at[t % 2], [d_lo, ci])
            i1 = plsc.load_gather(i_t.at[t % 2], [d_hi, ci])
            s = jnp.sum(u0 * i0 + u1 * i1)
            out = jnp.where(lane == t, s, out)
        sl = pl.ds(g * L, L)
        x = out + ub_v[sl] + ib_v[sl]
        out_v[sl] = 4.0 / (1.0 + jnp.exp(-x)) + 1.0
        return carry

    lax.fori_loop(0, ngroups, group, 0)

    for c in bias_copies:
        c.wait()
    pltpu.sync_copy(out_v, out_hbm.at[pl.ds(wid * bpw, bpw)])


def kernel(user, item, user_emb_w, item_emb_w, user_bias_w, item_bias_w):
    batch = user.shape[0]
    d = user_emb_w.shape[1]
    bpw = batch // NW
    nchunk = bpw // CHUNK
    assert bpw * NW == batch and nchunk * CHUNK == bpw

    mesh = plsc.VectorSubcoreMesh(
        core_axis_name="c", subcore_axis_name="s",
        num_cores=NC, num_subcores=NS)
    f = pl.kernel(
        _body,
        out_type=jax.ShapeDtypeStruct((batch,), jnp.float32),
        mesh=mesh,
        compiler_params=pltpu.CompilerParams(
            needs_layout_passes=False, use_tc_tiling_on_sc=True),
        scratch_types=[
            pltpu.VMEM((nchunk, CHUNK), jnp.int32),
            pltpu.VMEM((nchunk, CHUNK), jnp.int32),
            pltpu.VMEM((2, d, CHUNK), jnp.float32),
            pltpu.VMEM((2, d, CHUNK), jnp.float32),
            pltpu.VMEM((bpw,), jnp.float32),
            pltpu.VMEM((bpw,), jnp.float32),
            pltpu.VMEM((bpw,), jnp.float32),
            pltpu.SemaphoreType.DMA,
            pltpu.SemaphoreType.DMA,
            pltpu.SemaphoreType.DMA,
        ],
    )
    user2 = user.reshape(batch // CHUNK, CHUNK)
    item2 = item.reshape(batch // CHUNK, CHUNK)
    uet = jnp.swapaxes(user_emb_w, 0, 1)
    iet = jnp.swapaxes(item_emb_w, 0, 1)
    ub1 = user_bias_w.reshape(-1)
    ib1 = item_bias_w.reshape(-1)
    return f(user2, item2, uet, iet, ub1, ib1)


# speedup vs baseline: 2.8401x; 1.2361x over previous
import jax
import jax.numpy as jnp
from jax import lax
from jax.experimental import pallas as pl
from jax.experimental.pallas import tpu as pltpu
from jax.experimental.pallas import tpu_sc as plsc

NC = 2
NS = 16
NW = NC * NS
L = 16
CHUNK = 128
GPC = CHUNK // L


def _body(user_hbm, item_hbm, ue_hbm, ie_hbm, ub_hbm, ib_hbm, out_hbm,
          idx_vu, idx_vi, u_t, i_t, ub_v, ib_v, out_v, sem_u, sem_i, sem_b):
    nchunk, _ = idx_vu.shape
    bpw = nchunk * CHUNK
    d = u_t.shape[1]
    ngroups = bpw // L
    wid = lax.axis_index("s") * NC + lax.axis_index("c")
    base_chunk = wid * nchunk

    pltpu.sync_copy(user_hbm.at[pl.ds(base_chunk, nchunk)], idx_vu)
    pltpu.sync_copy(item_hbm.at[pl.ds(base_chunk, nchunk)], idx_vi)
    bias_copies = []
    for j in range(nchunk):
        sl = pl.ds(j * CHUNK, CHUNK)
        bias_copies.append(
            pltpu.async_copy(ub_hbm.at[idx_vu.at[j]], ub_v.at[sl], sem_b))
        bias_copies.append(
            pltpu.async_copy(ib_hbm.at[idx_vi.at[j]], ib_v.at[sl], sem_b))

    def load_group_idx(g):
        j = g // GPC
        k = (g % GPC) * L
        return idx_vu[j, pl.ds(k, L)], idx_vi[j, pl.ds(k, L)]

    def fire(ru, ri, buf):
        ctu = pl.multiple_of((ru // CHUNK) * CHUNK, CHUNK)
        cti = pl.multiple_of((ri // CHUNK) * CHUNK, CHUNK)
        pltpu.async_copy(ue_hbm.at[:, pl.ds(ctu, CHUNK)], u_t.at[buf], sem_u)
        pltpu.async_copy(ie_hbm.at[:, pl.ds(cti, CHUNK)], i_t.at[buf], sem_i)

    def drain(buf):
        pltpu.make_async_copy(
            ue_hbm.at[:, pl.ds(0, CHUNK)], u_t.at[buf], sem_u).wait()
        pltpu.make_async_copy(
            ie_hbm.at[:, pl.ds(0, CHUNK)], i_t.at[buf], sem_i).wait()

    lane = lax.broadcasted_iota(jnp.int32, (L,), 0)
    d_lo = lax.broadcasted_iota(jnp.int32, (L,), 0)
    d_hi = d_lo + L

    NBUF = 4
    iv_u0, iv_i0 = load_group_idx(0)
    for p in range(NBUF - 1):
        fire(iv_u0[p], iv_i0[p], p)

    def group(g, carry):
        iv_u, iv_i = load_group_idx(g)
        out = jnp.zeros((L,), jnp.float32)
        for t in range(L):
            if t < L - (NBUF - 1):
                fire(iv_u[t + NBUF - 1], iv_i[t + NBUF - 1],
                     (t + NBUF - 1) % NBUF)
            else:
                @pl.when(g + 1 < ngroups)
                def _():
                    iv_un, iv_in = load_group_idx(g + 1)
                    tn = t + NBUF - 1 - L
                    fire(iv_un[tn], iv_in[tn], (t + NBUF - 1) % NBUF)
            drain(t % NBUF)
            cu = jnp.full((L,), iv_u[t] % CHUNK, jnp.int32)
            ci = jnp.full((L,), iv_i[t] % CHUNK, jnp.int32)
            u0 = plsc.load_gather(u_t.at[t % NBUF], [d_lo, cu])
            u1 = plsc.load_gather(u_t.at[t % NBUF], [d_hi, cu])
            i0 = plsc.load_gather(i_t.at[t % NBUF], [d_lo, ci])
            i1 = plsc.load_gather(i_t.at[t % NBUF], [d_hi, ci])
            s = jnp.sum(u0 * i0 + u1 * i1)
            out = jnp.where(lane == t, s, out)
        sl = pl.ds(g * L, L)
        x = out + ub_v[sl] + ib_v[sl]
        out_v[sl] = 4.0 / (1.0 + jnp.exp(-x)) + 1.0
        return carry

    lax.fori_loop(0, ngroups, group, 0)

    for c in bias_copies:
        c.wait()
    pltpu.sync_copy(out_v, out_hbm.at[pl.ds(wid * bpw, bpw)])


def kernel(user, item, user_emb_w, item_emb_w, user_bias_w, item_bias_w):
    batch = user.shape[0]
    d = user_emb_w.shape[1]
    bpw = batch // NW
    nchunk = bpw // CHUNK
    assert bpw * NW == batch and nchunk * CHUNK == bpw

    mesh = plsc.VectorSubcoreMesh(
        core_axis_name="c", subcore_axis_name="s",
        num_cores=NC, num_subcores=NS)
    f = pl.kernel(
        _body,
        out_type=jax.ShapeDtypeStruct((batch,), jnp.float32),
        mesh=mesh,
        compiler_params=pltpu.CompilerParams(
            needs_layout_passes=False, use_tc_tiling_on_sc=True),
        scratch_types=[
            pltpu.VMEM((nchunk, CHUNK), jnp.int32),
            pltpu.VMEM((nchunk, CHUNK), jnp.int32),
            pltpu.VMEM((4, d, CHUNK), jnp.float32),
            pltpu.VMEM((4, d, CHUNK), jnp.float32),
            pltpu.VMEM((bpw,), jnp.float32),
            pltpu.VMEM((bpw,), jnp.float32),
            pltpu.VMEM((bpw,), jnp.float32),
            pltpu.SemaphoreType.DMA,
            pltpu.SemaphoreType.DMA,
            pltpu.SemaphoreType.DMA,
        ],
    )
    user2 = user.reshape(batch // CHUNK, CHUNK)
    item2 = item.reshape(batch // CHUNK, CHUNK)
    uet = jnp.swapaxes(user_emb_w, 0, 1)
    iet = jnp.swapaxes(item_emb_w, 0, 1)
    ub1 = user_bias_w.reshape(-1)
    ib1 = item_bias_w.reshape(-1)
    return f(user2, item2, uet, iet, ub1, ib1)


# speedup vs baseline: 3.1246x; 1.1002x over previous
import jax
import jax.numpy as jnp
from jax import lax
from jax.experimental import pallas as pl
from jax.experimental.pallas import tpu as pltpu
from jax.experimental.pallas import tpu_sc as plsc

NC = 2
NS = 16
NW = NC * NS
L = 16
CHUNK = 128
GPC = CHUNK // L


def _body(user_hbm, item_hbm, ue_hbm, ie_hbm, ub_hbm, ib_hbm, out_hbm,
          idx_vu, idx_vi, u_t, i_t, ub_v, ib_v, out_v, sem_u, sem_i, sem_b):
    nchunk, _ = idx_vu.shape
    bpw = nchunk * CHUNK
    d = u_t.shape[1]
    ngroups = bpw // L
    wid = lax.axis_index("s") * NC + lax.axis_index("c")
    base_chunk = wid * nchunk

    pltpu.sync_copy(user_hbm.at[pl.ds(base_chunk, nchunk)], idx_vu)
    pltpu.sync_copy(item_hbm.at[pl.ds(base_chunk, nchunk)], idx_vi)
    bias_copies = []
    for j in range(nchunk):
        sl = pl.ds(j * CHUNK, CHUNK)
        bias_copies.append(
            pltpu.async_copy(ub_hbm.at[idx_vu.at[j]], ub_v.at[sl], sem_b))
        bias_copies.append(
            pltpu.async_copy(ib_hbm.at[idx_vi.at[j]], ib_v.at[sl], sem_b))

    def load_group_idx(g):
        j = g // GPC
        k = (g % GPC) * L
        return idx_vu[j, pl.ds(k, L)], idx_vi[j, pl.ds(k, L)]

    def fire(ru, ri, buf):
        ctu = pl.multiple_of((ru // CHUNK) * CHUNK, CHUNK)
        cti = pl.multiple_of((ri // CHUNK) * CHUNK, CHUNK)
        pltpu.async_copy(ue_hbm.at[:, pl.ds(ctu, CHUNK)], u_t.at[buf], sem_u)
        pltpu.async_copy(ie_hbm.at[:, pl.ds(cti, CHUNK)], i_t.at[buf], sem_i)

    def drain(buf):
        pltpu.make_async_copy(
            ue_hbm.at[:, pl.ds(0, CHUNK)], u_t.at[buf], sem_u).wait()
        pltpu.make_async_copy(
            ie_hbm.at[:, pl.ds(0, CHUNK)], i_t.at[buf], sem_i).wait()

    lane = lax.broadcasted_iota(jnp.int32, (L,), 0)
    d_lo = lax.broadcasted_iota(jnp.int32, (L,), 0)
    d_hi = d_lo + L

    NBUF = 8
    iv_u0, iv_i0 = load_group_idx(0)
    for p in range(NBUF - 1):
        fire(iv_u0[p], iv_i0[p], p)

    def group(g, carry):
        iv_u, iv_i = load_group_idx(g)
        out = jnp.zeros((L,), jnp.float32)
        for t in range(L):
            if t < L - (NBUF - 1):
                fire(iv_u[t + NBUF - 1], iv_i[t + NBUF - 1],
                     (t + NBUF - 1) % NBUF)
            else:
                @pl.when(g + 1 < ngroups)
                def _():
                    iv_un, iv_in = load_group_idx(g + 1)
                    tn = t + NBUF - 1 - L
                    fire(iv_un[tn], iv_in[tn], (t + NBUF - 1) % NBUF)
            drain(t % NBUF)
            cu = jnp.full((L,), iv_u[t] % CHUNK, jnp.int32)
            ci = jnp.full((L,), iv_i[t] % CHUNK, jnp.int32)
            u0 = plsc.load_gather(u_t.at[t % NBUF], [d_lo, cu])
            u1 = plsc.load_gather(u_t.at[t % NBUF], [d_hi, cu])
            i0 = plsc.load_gather(i_t.at[t % NBUF], [d_lo, ci])
            i1 = plsc.load_gather(i_t.at[t % NBUF], [d_hi, ci])
            s = jnp.sum(u0 * i0 + u1 * i1)
            out = jnp.where(lane == t, s, out)
        sl = pl.ds(g * L, L)
        x = out + ub_v[sl] + ib_v[sl]
        out_v[sl] = 4.0 / (1.0 + jnp.exp(-x)) + 1.0
        return carry

    lax.fori_loop(0, ngroups, group, 0)

    for c in bias_copies:
        c.wait()
    pltpu.sync_copy(out_v, out_hbm.at[pl.ds(wid * bpw, bpw)])


def kernel(user, item, user_emb_w, item_emb_w, user_bias_w, item_bias_w):
    batch = user.shape[0]
    d = user_emb_w.shape[1]
    bpw = batch // NW
    nchunk = bpw // CHUNK
    assert bpw * NW == batch and nchunk * CHUNK == bpw

    mesh = plsc.VectorSubcoreMesh(
        core_axis_name="c", subcore_axis_name="s",
        num_cores=NC, num_subcores=NS)
    f = pl.kernel(
        _body,
        out_type=jax.ShapeDtypeStruct((batch,), jnp.float32),
        mesh=mesh,
        compiler_params=pltpu.CompilerParams(
            needs_layout_passes=False, use_tc_tiling_on_sc=True),
        scratch_types=[
            pltpu.VMEM((nchunk, CHUNK), jnp.int32),
            pltpu.VMEM((nchunk, CHUNK), jnp.int32),
            pltpu.VMEM((8, d, CHUNK), jnp.float32),
            pltpu.VMEM((8, d, CHUNK), jnp.float32),
            pltpu.VMEM((bpw,), jnp.float32),
            pltpu.VMEM((bpw,), jnp.float32),
            pltpu.VMEM((bpw,), jnp.float32),
            pltpu.SemaphoreType.DMA,
            pltpu.SemaphoreType.DMA,
            pltpu.SemaphoreType.DMA,
        ],
    )
    user2 = user.reshape(batch // CHUNK, CHUNK)
    item2 = item.reshape(batch // CHUNK, CHUNK)
    uet = jnp.swapaxes(user_emb_w, 0, 1)
    iet = jnp.swapaxes(item_emb_w, 0, 1)
    ub1 = user_bias_w.reshape(-1)
    ib1 = item_bias_w.reshape(-1)
    return f(user2, item2, uet, iet, ub1, ib1)
